# baseline (speedup 1.0000x reference)
import functools

import jax
import jax.numpy as jnp
from jax import lax
from jax.experimental import pallas as pl
from jax.experimental.pallas import tpu as pltpu
from jax.experimental.pallas import tpu_sc as plsc

_B = 16384
_D = 32
_CHUNK = 2
_SLOTS = 12


@functools.cache
def _build():
    info = plsc.get_sparse_core_info()
    nw = info.num_cores * info.num_subcores
    b_per_w = _B // nw
    n_pairs = b_per_w // (2 * _CHUNK)
    mesh = plsc.VectorSubcoreMesh(core_axis_name="c", subcore_axis_name="s")

    @functools.partial(
        pl.kernel,
        mesh=mesh,
        compiler_params=pltpu.CompilerParams(needs_layout_passes=False),
        out_type=jax.ShapeDtypeStruct((_D, _B), jnp.float32),
        scratch_types=[
            pltpu.VMEM((b_per_w,), jnp.int32),
            pltpu.VMEM((_SLOTS, _CHUNK, _D, 128), jnp.float32),
            pltpu.VMEM((_D, b_per_w), jnp.float32),
            pltpu.SemaphoreType.DMA,
        ] + [pltpu.SemaphoreType.DMA] * _SLOTS,
    )
    def gather_kernel(
        idx_hbm, tab_hbm, out_hbm, idx_v, blk_v, rows_v, sem_i, *sems
    ):
        wid = lax.axis_index("s") * info.num_cores + lax.axis_index("c")
        base = pl.multiple_of(wid * b_per_w, 128)
        pltpu.async_copy(idx_hbm.at[pl.ds(base, b_per_w)], idx_v, sem_i).wait()
        lane = lax.iota(jnp.int32, 16)
        n_chunks = b_per_w // _CHUNK
        per_cv = 16 // _CHUNK

        def fire(k, buf, sem):
            cv = idx_v[pl.ds((k // per_cv) * 16, 16)]
            half = k % per_cv
            for kk in range(_CHUNK):
                c = jnp.sum(jnp.where(lane == half * _CHUNK + kk, cv, 0))
                t = pl.multiple_of(c - c % 128, 128)
                pltpu.async_copy(
                    tab_hbm.at[:, pl.ds(t, 128)], blk_v.at[buf, kk], sem
                )

        def drain(sem):
            for kk in range(_CHUNK):
                pltpu.make_async_copy(
                    tab_hbm.at[:, pl.ds(0, 128)], blk_v.at[0, kk], sem
                ).wait()

        def extract(k, buf):
            cv = idx_v[pl.ds((k // per_cv) * 16, 16)]
            half = k % per_cv
            jbase = k * _CHUNK
            for kk in range(_CHUNK):
                c = jnp.sum(jnp.where(lane == half * _CHUNK + kk, cv, 0))
                r = c % 128
                kcol = jnp.full((16,), kk, dtype=jnp.int32)
                bcol = jnp.full((16,), buf, dtype=jnp.int32)
                rcol = jnp.full((16,), r, dtype=jnp.int32)
                jcol = jnp.full((16,), jbase + kk, dtype=jnp.int32)
                lo = plsc.load_gather(blk_v, [bcol, kcol, lane, rcol])
                hi = plsc.load_gather(blk_v, [bcol, kcol, lane + 16, rcol])
                plsc.store_scatter(rows_v, [lane, jcol], lo)
                plsc.store_scatter(rows_v, [lane + 16, jcol], hi)

        for pre in range(_SLOTS - 1):
            fire(pre, pre, sems[pre])

        @pl.loop(0, n_chunks)
        def _chunk(k):
            for s in range(_SLOTS):
                @pl.when((k % _SLOTS) == s)
                def _slot(s=s):
                    nxt = (s + _SLOTS - 1) % _SLOTS

                    @pl.when(k + _SLOTS - 1 < n_chunks)
                    def _pf():
                        fire(k + _SLOTS - 1, nxt, sems[nxt])

                    drain(sems[s])
                    extract(k, s)

        pltpu.sync_copy(rows_v, out_hbm.at[:, pl.ds(base, b_per_w)])

    return gather_kernel


def kernel(beta, Emb):
    out_t = _build()(beta.astype(jnp.int32), Emb.T)
    return out_t.T


# speedup vs baseline: 1.0201x; 1.0201x over previous
import functools

import jax
import jax.numpy as jnp
from jax import lax
from jax.experimental import pallas as pl
from jax.experimental.pallas import tpu as pltpu
from jax.experimental.pallas import tpu_sc as plsc

_B = 16384
_D = 32
_CHUNK = 4
_SLOTS = 6


@functools.cache
def _build():
    info = plsc.get_sparse_core_info()
    nw = info.num_cores * info.num_subcores
    b_per_w = _B // nw
    mesh = plsc.VectorSubcoreMesh(core_axis_name="c", subcore_axis_name="s")

    @functools.partial(
        pl.kernel,
        mesh=mesh,
        compiler_params=pltpu.CompilerParams(needs_layout_passes=False),
        out_type=jax.ShapeDtypeStruct((_D, _B), jnp.float32),
        scratch_types=[
            pltpu.VMEM((b_per_w,), jnp.int32),
            pltpu.VMEM((_SLOTS, _CHUNK, _D, 128), jnp.float32),
            pltpu.VMEM((_D, b_per_w), jnp.float32),
            pltpu.SemaphoreType.DMA,
        ] + [pltpu.SemaphoreType.DMA] * _SLOTS,
    )
    def gather_kernel(
        idx_hbm, tab_hbm, out_hbm, idx_v, blk_v, rows_v, sem_i, *sems
    ):
        wid = lax.axis_index("s") * info.num_cores + lax.axis_index("c")
        base = pl.multiple_of(wid * b_per_w, 128)
        pltpu.async_copy(idx_hbm.at[pl.ds(base, b_per_w)], idx_v, sem_i).wait()
        lane = lax.iota(jnp.int32, 16)
        n_chunks = b_per_w // _CHUNK
        per_cv = 16 // _CHUNK

        def fire(k, buf, sem):
            cv = idx_v[pl.ds((k // per_cv) * 16, 16)]
            half = k % per_cv
            for kk in range(_CHUNK):
                c = jnp.sum(jnp.where(lane == half * _CHUNK + kk, cv, 0))
                t = pl.multiple_of(c - c % 128, 128)
                pltpu.async_copy(
                    tab_hbm.at[:, pl.ds(t, 128)], blk_v.at[buf, kk], sem
                )

        def drain(sem):
            for kk in range(_CHUNK):
                pltpu.make_async_copy(
                    tab_hbm.at[:, pl.ds(0, 128)], blk_v.at[0, kk], sem
                ).wait()

        def extract(k, buf):
            cv = idx_v[pl.ds((k // per_cv) * 16, 16)]
            half = k % per_cv
            jbase = k * _CHUNK
            for kk in range(_CHUNK):
                c = jnp.sum(jnp.where(lane == half * _CHUNK + kk, cv, 0))
                r = c % 128
                kcol = jnp.full((16,), kk, dtype=jnp.int32)
                bcol = jnp.full((16,), buf, dtype=jnp.int32)
                rcol = jnp.full((16,), r, dtype=jnp.int32)
                jcol = jnp.full((16,), jbase + kk, dtype=jnp.int32)
                lo = plsc.load_gather(blk_v, [bcol, kcol, lane, rcol])
                hi = plsc.load_gather(blk_v, [bcol, kcol, lane + 16, rcol])
                plsc.store_scatter(rows_v, [lane, jcol], lo)
                plsc.store_scatter(rows_v, [lane + 16, jcol], hi)

        for pre in range(_SLOTS - 1):
            fire(pre, pre, sems[pre])

        @pl.loop(0, n_chunks)
        def _chunk(k):
            for s in range(_SLOTS):
                @pl.when((k % _SLOTS) == s)
                def _slot(s=s):
                    nxt = (s + _SLOTS - 1) % _SLOTS

                    @pl.when(k + _SLOTS - 1 < n_chunks)
                    def _pf():
                        fire(k + _SLOTS - 1, nxt, sems[nxt])

                    drain(sems[s])
                    extract(k, s)

        pltpu.sync_copy(rows_v, out_hbm.at[:, pl.ds(base, b_per_w)])

    return gather_kernel


def kernel(beta, Emb):
    out_t = _build()(beta.astype(jnp.int32), Emb.T)
    return out_t.T


# speedup vs baseline: 1.0233x; 1.0031x over previous
import functools

import jax
import jax.numpy as jnp
from jax import lax
from jax.experimental import pallas as pl
from jax.experimental.pallas import tpu as pltpu
from jax.experimental.pallas import tpu_sc as plsc

_B = 16384
_D = 32
_CHUNK = 4
_SLOTS = 6


@functools.cache
def _build():
    info = plsc.get_sparse_core_info()
    nw = info.num_cores * info.num_subcores
    b_per_w = _B // nw
    mesh = plsc.VectorSubcoreMesh(core_axis_name="c", subcore_axis_name="s")

    @functools.partial(
        pl.kernel,
        mesh=mesh,
        compiler_params=pltpu.CompilerParams(needs_layout_passes=False),
        out_type=jax.ShapeDtypeStruct((_D, _B), jnp.float32),
        scratch_types=[
            pltpu.VMEM((b_per_w + 16,), jnp.int32),
            pltpu.VMEM((_SLOTS, _CHUNK, _D, 128), jnp.float32),
            pltpu.VMEM((_D, b_per_w), jnp.float32),
            pltpu.SemaphoreType.DMA,
        ] + [pltpu.SemaphoreType.DMA] * _SLOTS,
    )
    def gather_kernel(
        idx_hbm, tab_hbm, out_hbm, idx_v, blk_v, rows_v, sem_i, *sems
    ):
        wid = lax.axis_index("s") * info.num_cores + lax.axis_index("c")
        base = pl.multiple_of(wid * b_per_w, 128)
        pltpu.async_copy(
            idx_hbm.at[pl.ds(base, b_per_w)], idx_v.at[pl.ds(0, b_per_w)], sem_i
        ).wait()
        lane = lax.iota(jnp.int32, 16)
        n_chunks = b_per_w // _CHUNK

        def fire(k, buf, sem):
            cv = idx_v[pl.ds(k * _CHUNK, 16)]
            for kk in range(_CHUNK):
                c = cv[kk]
                t = pl.multiple_of(c - c % 128, 128)
                pltpu.async_copy(
                    tab_hbm.at[:, pl.ds(t, 128)], blk_v.at[buf, kk], sem
                )

        def drain(sem):
            for kk in range(_CHUNK):
                pltpu.make_async_copy(
                    tab_hbm.at[:, pl.ds(0, 128)], blk_v.at[0, kk], sem
                ).wait()

        def extract(k, buf):
            cv = idx_v[pl.ds(k * _CHUNK, 16)]
            jbase = k * _CHUNK
            for kk in range(_CHUNK):
                c = cv[kk]
                r = c % 128
                kcol = jnp.full((16,), kk, dtype=jnp.int32)
                bcol = jnp.full((16,), buf, dtype=jnp.int32)
                rcol = jnp.full((16,), r, dtype=jnp.int32)
                jcol = jnp.full((16,), jbase + kk, dtype=jnp.int32)
                lo = plsc.load_gather(blk_v, [bcol, kcol, lane, rcol])
                hi = plsc.load_gather(blk_v, [bcol, kcol, lane + 16, rcol])
                plsc.store_scatter(rows_v, [lane, jcol], lo)
                plsc.store_scatter(rows_v, [lane + 16, jcol], hi)

        for pre in range(_SLOTS - 1):
            fire(pre, pre, sems[pre])

        @pl.loop(0, n_chunks)
        def _chunk(k):
            for s in range(_SLOTS):
                @pl.when((k % _SLOTS) == s)
                def _slot(s=s):
                    nxt = (s + _SLOTS - 1) % _SLOTS

                    @pl.when(k + _SLOTS - 1 < n_chunks)
                    def _pf():
                        fire(k + _SLOTS - 1, nxt, sems[nxt])

                    drain(sems[s])
                    extract(k, s)

        pltpu.sync_copy(rows_v, out_hbm.at[:, pl.ds(base, b_per_w)])

    return gather_kernel


def kernel(beta, Emb):
    out_t = _build()(beta.astype(jnp.int32), Emb.T)
    return out_t.T
